# baseline (speedup 1.0000x reference)
import jax
import jax.numpy as jnp
from jax.experimental import pallas as pl
from jax.experimental.pallas import tpu as pltpu


def _copy_body(src_hbm, dst_hbm, sem):
    copy = pltpu.make_async_copy(src_hbm, dst_hbm.at[0], sem)
    copy.start()
    copy.wait()


def kernel(x, pe_weight):
    seq_len = x.shape[1]
    hidden = pe_weight.shape[1]
    return pl.pallas_call(
        _copy_body,
        out_shape=jax.ShapeDtypeStruct((1, seq_len, hidden), pe_weight.dtype),
        in_specs=[pl.BlockSpec(memory_space=pl.ANY)],
        out_specs=pl.BlockSpec(memory_space=pl.ANY),
        scratch_shapes=[pltpu.SemaphoreType.DMA],
    )(pe_weight)
